# Initial kernel scaffold; baseline (speedup 1.0000x reference)
import jax
import jax.numpy as jnp
from jax.experimental import pallas as pl


def kernel(x, edge_index, pre_W, pre_b, fc1_W, fc1_b, fc2_W, fc2_b, gcn_W, gcn_b, dense_W, dense_b, f1_W, f1_b, f2_W, f2_b):
    raise NotImplementedError("write your pallas kernel here")


# speedup vs baseline: 19.5305x; 19.5305x over previous
import functools

import jax
import jax.numpy as jnp
from jax import lax
from jax.experimental import pallas as pl
from jax.experimental.pallas import tpu as pltpu
from jax.experimental.pallas import tpu_sc as plsc

N = 100000
E = 1600000
D = 32
HALF = 16
NC = 2
NS = 16

BLK = 3136
GRID = 32
ACCR = BLK * GRID
DUMMY = N

G_EDGES = 1024
EPAD = 1605632
CROWS = EPAD // 128
ROWS_PER_TILE = CROWS // NS
GROUPS_SCATTER = ROWS_PER_TILE // 8
ROWS_PER_TILE_DEG = CROWS // (NS * NC)
GROUPS_DEG = ROWS_PER_TILE_DEG // 8
OUT_SLICE = ACCR // NS
ZROWS = OUT_SLICE // 16

_mesh = plsc.VectorSubcoreMesh(core_axis_name="c", subcore_axis_name="s")
_sc_params = pltpu.CompilerParams(use_tc_tiling_on_sc=False)


def _fill(ref, rows, value):
    v = jnp.full((16,), value, jnp.float32)

    def body(i, _):
        ref[i, :] = v
        return 0

    lax.fori_loop(0, rows, body, 0)


def _fill3(ref, d0, d1, value):
    v = jnp.full((16,), value, jnp.float32)

    def body(i, _):
        ref[i // d1, i % d1, :] = v
        return 0

    lax.fori_loop(0, d0 * d1, body, 0)


@functools.partial(
    pl.kernel,
    out_type=jax.ShapeDtypeStruct((NC, ACCR, HALF), jnp.float32),
    mesh=_mesh,
    scratch_types=dict(
        didx=pltpu.VMEM((8, 128), jnp.int32),
        ones=pltpu.VMEM((8, 128, HALF), jnp.float32),
        stage=pltpu.VMEM((ZROWS, HALF), jnp.float32),
        acc=pltpu.VMEM_SHARED((ACCR, HALF), jnp.float32),
    ),
    compiler_params=_sc_params,
)
def _sc_degree(dst_hbm, out_hbm, didx, ones, stage, acc):
    c = lax.axis_index("c")
    s = lax.axis_index("s")
    w = s * NC + c

    _fill(stage, ZROWS, 0.0)
    _fill3(ones, 8, 128, 1.0)
    for k in range(16):
        pltpu.sync_copy(stage, acc.at[pl.ds(s * OUT_SLICE + k * ZROWS, ZROWS)])
    plsc.subcore_barrier()

    def body(g, _):
        base = w * ROWS_PER_TILE_DEG + g * 8
        pltpu.sync_copy(dst_hbm.at[pl.ds(base, 8), :], didx)
        for j in range(8):
            pltpu.sync_copy(ones.at[j], acc.at[didx.at[j]], add=True)
        return 0

    lax.fori_loop(0, GROUPS_DEG, body, 0)
    plsc.subcore_barrier()

    for k in range(16):
        off = s * OUT_SLICE + k * ZROWS
        pltpu.sync_copy(acc.at[pl.ds(off, ZROWS)], stage)
        pltpu.sync_copy(stage, out_hbm.at[c, pl.ds(off, ZROWS), :])


@functools.partial(
    pl.kernel,
    out_type=jax.ShapeDtypeStruct((NC, ACCR, HALF), jnp.float32),
    mesh=_mesh,
    scratch_types=dict(
        sidx=pltpu.VMEM((2, 8, 128), jnp.int32),
        didx=pltpu.VMEM((2, 8, 128), jnp.int32),
        rows=pltpu.VMEM((8, 128, HALF), jnp.float32),
        stage=pltpu.VMEM((ZROWS, HALF), jnp.float32),
        acc=pltpu.VMEM_SHARED((ACCR, HALF), jnp.float32),
        gsem=pltpu.SemaphoreType.DMA((8,)),
        ssem=pltpu.SemaphoreType.DMA((8,)),
    ),
    compiler_params=_sc_params,
)
def _sc_scatter(src_hbm, dst_hbm, g_hbm, out_hbm, sidx, didx, rows, stage, acc,
                gsem, ssem):
    c = lax.axis_index("c")
    s = lax.axis_index("s")

    _fill(stage, ZROWS, 0.0)
    for k in range(16):
        pltpu.sync_copy(stage, acc.at[pl.ds(s * OUT_SLICE + k * ZROWS, ZROWS)])
    plsc.subcore_barrier()

    def _wait_scatter(j):
        pltpu.make_async_copy(rows.at[j], acc.at[pl.ds(0, 128)], ssem.at[j]).wait()

    def body(g, _):
        par = lax.rem(g, 2)
        base = s * ROWS_PER_TILE + g * 8
        pltpu.sync_copy(src_hbm.at[c, pl.ds(base, 8), :], sidx.at[par])
        pltpu.sync_copy(dst_hbm.at[pl.ds(base, 8), :], didx.at[par])

        @pl.when(g > 0)
        def _():
            for j in range(8):
                _wait_scatter(j)

        for j in range(8):
            pltpu.async_copy(g_hbm.at[sidx.at[par, j]], rows.at[j], gsem.at[j])
        for j in range(8):
            pltpu.make_async_copy(g_hbm.at[pl.ds(0, 128)], rows.at[j],
                                  gsem.at[j]).wait()
            pltpu.async_copy(rows.at[j], acc.at[didx.at[par, j]], ssem.at[j],
                             add=True)
        return 0

    lax.fori_loop(0, GROUPS_SCATTER, body, 0)
    for j in range(8):
        _wait_scatter_final = pltpu.make_async_copy(
            rows.at[j], acc.at[pl.ds(0, 128)], ssem.at[j])
        _wait_scatter_final.wait()
    plsc.subcore_barrier()

    for k in range(16):
        off = s * OUT_SLICE + k * ZROWS
        pltpu.sync_copy(acc.at[pl.ds(off, ZROWS)], stage)
        pltpu.sync_copy(stage, out_hbm.at[c, pl.ds(off, ZROWS), :])


def _dot(a, b):
    return jax.lax.dot_general(a, b, (((1,), (0,)), ((), ())),
                               preferred_element_type=jnp.float32)


def _wspec(shape):
    return pl.BlockSpec(shape, lambda i: tuple(0 for _ in shape))


def _pre_body(x_ref, deg_ref, preW, preb, fc1W, fc1b, fc2W, fc2b, gcnWA,
              gcnWB, denseWt, denseb, f1Wt, f1b,
              dinv_ref, based_ref, basef_ref, g2_ref):
    x1 = jax.nn.sigmoid(_dot(x_ref[...], preW[...]) + preb[...])
    ni = jax.nn.relu(_dot(x1, fc1W[...]) + fc1b[...])
    h0 = jax.nn.relu(_dot(x1, fc2W[...]) + fc2b[...])
    db = deg_ref[...]
    deg = db[0, :, 0:1] + db[1, :, 0:1] + 1.0
    dinv = lax.rsqrt(deg)
    dinv_ref[...] = dinv
    based_ref[...] = _dot(ni, denseWt[...]) + denseb[...]
    basef_ref[...] = _dot(ni, f1Wt[...]) + f1b[...]
    g2_ref[0] = _dot(h0, gcnWA[...]) * dinv
    g2_ref[1] = _dot(h0, gcnWB[...]) * dinv


def _layer_body(acc_ref, g_ref, dinv_ref, based_ref, gcnbA, gcnbB,
                denseWbA, denseWbB, gcnWA, gcnWB, gout_ref):
    a = acc_ref[...]
    gb = g_ref[...]
    dinv = dinv_ref[...]
    goA = jax.nn.relu(dinv * (a[0] + gb[0]) + gcnbA[...])
    goB = jax.nn.relu(dinv * (a[1] + gb[1]) + gcnbB[...])
    h = jax.nn.relu(_dot(goA, denseWbA[...]) + _dot(goB, denseWbB[...])
                    + based_ref[...])
    gout_ref[0] = _dot(h, gcnWA[...]) * dinv
    gout_ref[1] = _dot(h, gcnWB[...]) * dinv


def _final_body(acc_ref, g_ref, dinv_ref, based_ref, basef_ref, gcnbA, gcnbB,
                denseWbA, denseWbB, f1Wb, f2W, f2b, out_ref):
    a = acc_ref[...]
    gb = g_ref[...]
    dinv = dinv_ref[...]
    goA = jax.nn.relu(dinv * (a[0] + gb[0]) + gcnbA[...])
    goB = jax.nn.relu(dinv * (a[1] + gb[1]) + gcnbB[...])
    h = jax.nn.relu(_dot(goA, denseWbA[...]) + _dot(goB, denseWbB[...])
                    + based_ref[...])
    h2 = jax.nn.relu(_dot(h, f1Wb[...]) + basef_ref[...])
    out_ref[...] = _dot(h2, f2W[...]) + f2b[...]


_half_spec = pl.BlockSpec((NC, BLK, HALF), lambda i: (0, i, 0))
_row_spec = lambda w: pl.BlockSpec((BLK, w), lambda i: (i, 0))


def _k_pre(x, degout, preW, preb, fc1W, fc1b, fc2W, fc2b, gcnWA, gcnWB,
           denseWt, denseb, f1Wt, f1b):
    return pl.pallas_call(
        _pre_body,
        grid=(GRID,),
        in_specs=[
            _row_spec(6), _half_spec,
            _wspec((6, 10)), _wspec((1, 10)),
            _wspec((10, D)), _wspec((1, D)),
            _wspec((10, D)), _wspec((1, D)),
            _wspec((D, HALF)), _wspec((D, HALF)),
            _wspec((D, D)), _wspec((1, D)),
            _wspec((D, D)), _wspec((1, D)),
        ],
        out_specs=[_row_spec(1), _row_spec(D), _row_spec(D), _half_spec],
        out_shape=[
            jax.ShapeDtypeStruct((ACCR, 1), jnp.float32),
            jax.ShapeDtypeStruct((ACCR, D), jnp.float32),
            jax.ShapeDtypeStruct((ACCR, D), jnp.float32),
            jax.ShapeDtypeStruct((NC, ACCR, HALF), jnp.float32),
        ],
    )(x, degout, preW, preb, fc1W, fc1b, fc2W, fc2b, gcnWA, gcnWB, denseWt,
      denseb, f1Wt, f1b)


def _k_layer(acc2, g2, dinv, based, gcnbA, gcnbB, denseWbA, denseWbB,
             gcnWA, gcnWB):
    return pl.pallas_call(
        _layer_body,
        grid=(GRID,),
        in_specs=[
            _half_spec, _half_spec, _row_spec(1), _row_spec(D),
            _wspec((1, HALF)), _wspec((1, HALF)),
            _wspec((HALF, D)), _wspec((HALF, D)),
            _wspec((D, HALF)), _wspec((D, HALF)),
        ],
        out_specs=_half_spec,
        out_shape=jax.ShapeDtypeStruct((NC, ACCR, HALF), jnp.float32),
    )(acc2, g2, dinv, based, gcnbA, gcnbB, denseWbA, denseWbB, gcnWA, gcnWB)


def _k_final(acc2, g2, dinv, based, basef, gcnbA, gcnbB, denseWbA, denseWbB,
             f1Wb, f2W, f2b):
    return pl.pallas_call(
        _final_body,
        grid=(GRID,),
        in_specs=[
            _half_spec, _half_spec, _row_spec(1), _row_spec(D), _row_spec(D),
            _wspec((1, HALF)), _wspec((1, HALF)),
            _wspec((HALF, D)), _wspec((HALF, D)),
            _wspec((D, D)),
            _wspec((D, 2)), _wspec((1, 2)),
        ],
        out_specs=_row_spec(2),
        out_shape=jax.ShapeDtypeStruct((N, 2), jnp.float32),
    )(acc2, g2, dinv, based, basef, gcnbA, gcnbB, denseWbA, denseWbB,
      f1Wb, f2W, f2b)


def kernel(x, edge_index, pre_W, pre_b, fc1_W, fc1_b, fc2_W, fc2_b,
           gcn_W, gcn_b, dense_W, dense_b, f1_W, f1_b, f2_W, f2_b):
    src = edge_index[0]
    dst = edge_index[1]
    pad = EPAD - E
    src3 = jnp.concatenate([src, jnp.zeros((pad,), jnp.int32)]).reshape(CROWS, 128)
    dst3 = jnp.concatenate([dst, jnp.full((pad,), DUMMY, jnp.int32)]).reshape(CROWS, 128)
    src2 = jnp.stack([src3, src3 + ACCR])

    degout = _sc_degree(dst3)

    gcnWA = gcn_W[:, 0:HALF]
    gcnWB = gcn_W[:, HALF:D]
    gcnbA = gcn_b[0:HALF].reshape(1, -1)
    gcnbB = gcn_b[HALF:D].reshape(1, -1)
    denseWt = dense_W[0:D]
    denseWbA = dense_W[D:D + HALF]
    denseWbB = dense_W[D + HALF:2 * D]
    f1Wt = f1_W[0:D]
    f1Wb = f1_W[D:2 * D]

    dinv, based, basef, g2 = _k_pre(
        x, degout, pre_W, pre_b.reshape(1, -1), fc1_W, fc1_b.reshape(1, -1),
        fc2_W, fc2_b.reshape(1, -1), gcnWA, gcnWB, denseWt,
        dense_b.reshape(1, -1), f1Wt, f1_b.reshape(1, -1))

    for layer in range(6):
        acc2 = _sc_scatter(src2, dst3, g2.reshape(NC * ACCR, HALF))
        if layer < 5:
            g2 = _k_layer(acc2, g2, dinv, based, gcnbA, gcnbB, denseWbA,
                          denseWbB, gcnWA, gcnWB)
        else:
            out = _k_final(acc2, g2, dinv, based, basef, gcnbA, gcnbB,
                           denseWbA, denseWbB, f1Wb, f2_W,
                           f2_b.reshape(1, -1))
    return out


# speedup vs baseline: 28.6048x; 1.4646x over previous
import functools

import jax
import jax.numpy as jnp
from jax import lax
from jax.experimental import pallas as pl
from jax.experimental.pallas import tpu as pltpu
from jax.experimental.pallas import tpu_sc as plsc

N = 100000
E = 1600000
D = 32
HALF = 16
NC = 2
NS = 16

BLK = 3136
GRID = 32
ACCR = BLK * GRID
DUMMY = N

G_EDGES = 1024
EPAD = 1605632
CROWS = EPAD // 128
ROWS_PER_TILE = CROWS // NS
GROUPS_SCATTER = ROWS_PER_TILE // 8
ROWS_PER_TILE_DEG = CROWS // (NS * NC)
GROUPS_DEG = ROWS_PER_TILE_DEG // 8
OUT_SLICE = ACCR // NS
ZROWS = OUT_SLICE // 16

_mesh = plsc.VectorSubcoreMesh(core_axis_name="c", subcore_axis_name="s")
_sc_params = pltpu.CompilerParams(use_tc_tiling_on_sc=False)


def _fill(ref, rows, value):
    v = jnp.full((16,), value, jnp.float32)

    def body(i, _):
        ref[i, :] = v
        return 0

    lax.fori_loop(0, rows, body, 0)


def _fill3(ref, d0, d1, value):
    v = jnp.full((16,), value, jnp.float32)

    def body(i, _):
        ref[i // d1, i % d1, :] = v
        return 0

    lax.fori_loop(0, d0 * d1, body, 0)


@functools.partial(
    pl.kernel,
    out_type=jax.ShapeDtypeStruct((NC, ACCR, HALF), jnp.float32),
    mesh=_mesh,
    scratch_types=dict(
        didx=pltpu.VMEM((8, 128), jnp.int32),
        ones=pltpu.VMEM((8, 128, HALF), jnp.float32),
        stage=pltpu.VMEM((ZROWS, HALF), jnp.float32),
        acc=pltpu.VMEM_SHARED((ACCR, HALF), jnp.float32),
    ),
    compiler_params=_sc_params,
)
def _sc_degree(dst_hbm, out_hbm, didx, ones, stage, acc):
    c = lax.axis_index("c")
    s = lax.axis_index("s")
    w = s * NC + c

    _fill(stage, ZROWS, 0.0)
    _fill3(ones, 8, 128, 1.0)
    for k in range(16):
        pltpu.sync_copy(stage, acc.at[pl.ds(s * OUT_SLICE + k * ZROWS, ZROWS)])
    plsc.subcore_barrier()

    def body(g, _):
        base = w * ROWS_PER_TILE_DEG + g * 8
        pltpu.sync_copy(dst_hbm.at[pl.ds(base, 8), :], didx)
        for j in range(8):
            pltpu.sync_copy(ones.at[j], acc.at[didx.at[j]], add=True)
        return 0

    lax.fori_loop(0, GROUPS_DEG, body, 0)
    plsc.subcore_barrier()

    for k in range(16):
        off = s * OUT_SLICE + k * ZROWS
        pltpu.sync_copy(acc.at[pl.ds(off, ZROWS)], stage)
        pltpu.sync_copy(stage, out_hbm.at[c, pl.ds(off, ZROWS), :])


@functools.partial(
    pl.kernel,
    out_type=jax.ShapeDtypeStruct((NC, ACCR, HALF), jnp.float32),
    mesh=_mesh,
    scratch_types=dict(
        sidx=pltpu.VMEM((2, 8, 128), jnp.int32),
        didx=pltpu.VMEM((2, 8, 128), jnp.int32),
        rows=pltpu.VMEM((8, 128, HALF), jnp.float32),
        stage=pltpu.VMEM((ZROWS, HALF), jnp.float32),
        acc=pltpu.VMEM_SHARED((ACCR, HALF), jnp.float32),
        gsem=pltpu.SemaphoreType.DMA((8,)),
        ssem=pltpu.SemaphoreType.DMA((8,)),
    ),
    compiler_params=_sc_params,
)
def _sc_scatter(src_hbm, dst_hbm, g_hbm, out_hbm, sidx, didx, rows, stage, acc,
                gsem, ssem):
    c = lax.axis_index("c")
    s = lax.axis_index("s")

    _fill(stage, ZROWS, 0.0)
    for k in range(16):
        pltpu.sync_copy(stage, acc.at[pl.ds(s * OUT_SLICE + k * ZROWS, ZROWS)])
    plsc.subcore_barrier()

    def _wait_scatter(j):
        pltpu.make_async_copy(rows.at[j], acc.at[pl.ds(0, 128)], ssem.at[j]).wait()

    def body(g, _):
        par = lax.rem(g, 2)
        base = s * ROWS_PER_TILE + g * 8
        pltpu.sync_copy(src_hbm.at[c, pl.ds(base, 8), :], sidx.at[par])
        pltpu.sync_copy(dst_hbm.at[pl.ds(base, 8), :], didx.at[par])

        @pl.when(g > 0)
        def _():
            for j in range(8):
                _wait_scatter(j)

        for j in range(8):
            pltpu.async_copy(g_hbm.at[sidx.at[par, j]], rows.at[j], gsem.at[j])
        for j in range(8):
            pltpu.make_async_copy(g_hbm.at[pl.ds(0, 128)], rows.at[j],
                                  gsem.at[j]).wait()
            pltpu.async_copy(rows.at[j], acc.at[didx.at[par, j]], ssem.at[j],
                             add=True)
        return 0

    lax.fori_loop(0, GROUPS_SCATTER, body, 0)
    for j in range(8):
        _wait_scatter_final = pltpu.make_async_copy(
            rows.at[j], acc.at[pl.ds(0, 128)], ssem.at[j])
        _wait_scatter_final.wait()
    plsc.subcore_barrier()

    for k in range(16):
        off = s * OUT_SLICE + k * ZROWS
        pltpu.sync_copy(acc.at[pl.ds(off, ZROWS)], stage)
        pltpu.sync_copy(stage, out_hbm.at[c, pl.ds(off, ZROWS), :])


def _dot(a, b):
    return jax.lax.dot_general(a, b, (((1,), (0,)), ((), ())),
                               preferred_element_type=jnp.float32)


def _wspec(shape):
    return pl.BlockSpec(shape, lambda i: tuple(0 for _ in shape))


def _pre_body(x_ref, deg_ref, preW, preb, fc1W, fc1b, fc2W, fc2b, gcnWA,
              gcnWB, denseWt, denseb, f1Wt, f1b,
              dinv_ref, based_ref, basef_ref, g2_ref):
    x1 = jax.nn.sigmoid(_dot(x_ref[...], preW[...]) + preb[...])
    ni = jax.nn.relu(_dot(x1, fc1W[...]) + fc1b[...])
    h0 = jax.nn.relu(_dot(x1, fc2W[...]) + fc2b[...])
    db = deg_ref[...]
    deg = db[0, :, 0:1] + db[1, :, 0:1] + 1.0
    dinv = lax.rsqrt(deg)
    dinv_ref[...] = jnp.broadcast_to(dinv, (BLK, HALF))
    based_ref[...] = _dot(ni, denseWt[...]) + denseb[...]
    basef_ref[...] = _dot(ni, f1Wt[...]) + f1b[...]
    g2_ref[0] = _dot(h0, gcnWA[...]) * dinv
    g2_ref[1] = _dot(h0, gcnWB[...]) * dinv


def _layer_body(acc_ref, g_ref, dinv_ref, based_ref, biasA, biasB,
                bdDenseA, bdDenseB, bdGcnA, bdGcnB, gout_ref):
    a = acc_ref[...]
    gb = g_ref[...]
    dinv = dinv_ref[...]
    goA = jax.nn.relu(dinv * (a[0] + gb[0]) + biasA[...])
    goB = jax.nn.relu(dinv * (a[1] + gb[1]) + biasB[...])
    h = jax.nn.relu(_dot(goA, bdDenseA[...]) + _dot(goB, bdDenseB[...])
                    + based_ref[...])
    gout_ref[0] = _dot(h, bdGcnA[...]) * dinv
    gout_ref[1] = _dot(h, bdGcnB[...]) * dinv


def _final_body(acc_ref, g_ref, dinv_ref, based_ref, basef_ref, biasA, biasB,
                bdDenseA, bdDenseB, bdF1, bdF2, f2bt, out_ref):
    a = acc_ref[...]
    gb = g_ref[...]
    dinv = dinv_ref[...]
    goA = jax.nn.relu(dinv * (a[0] + gb[0]) + biasA[...])
    goB = jax.nn.relu(dinv * (a[1] + gb[1]) + biasB[...])
    h = jax.nn.relu(_dot(goA, bdDenseA[...]) + _dot(goB, bdDenseB[...])
                    + based_ref[...])
    h2 = jax.nn.relu(_dot(h, bdF1[...]) + basef_ref[...])
    out_ref[...] = _dot(h2, bdF2[...]) + f2bt[...]


PR = ACCR // 8
PBLK = BLK // 8
PGRID = GRID

_phalf_spec = pl.BlockSpec((NC, PBLK, 128), lambda i: (0, i, 0))
_prow_spec = lambda w: pl.BlockSpec((PBLK, w), lambda i: (i, 0))
_half_spec = pl.BlockSpec((NC, BLK, HALF), lambda i: (0, i, 0))
_row_spec = lambda w: pl.BlockSpec((BLK, w), lambda i: (i, 0))


def _k_pre(x, degout, preW, preb, fc1W, fc1b, fc2W, fc2b, gcnWA, gcnWB,
           denseWt, denseb, f1Wt, f1b):
    return pl.pallas_call(
        _pre_body,
        grid=(GRID,),
        in_specs=[
            _row_spec(6), _half_spec,
            _wspec((6, 10)), _wspec((1, 10)),
            _wspec((10, D)), _wspec((1, D)),
            _wspec((10, D)), _wspec((1, D)),
            _wspec((D, HALF)), _wspec((D, HALF)),
            _wspec((D, D)), _wspec((1, D)),
            _wspec((D, D)), _wspec((1, D)),
        ],
        out_specs=[_row_spec(HALF), _row_spec(D), _row_spec(D), _half_spec],
        out_shape=[
            jax.ShapeDtypeStruct((ACCR, HALF), jnp.float32),
            jax.ShapeDtypeStruct((ACCR, D), jnp.float32),
            jax.ShapeDtypeStruct((ACCR, D), jnp.float32),
            jax.ShapeDtypeStruct((NC, ACCR, HALF), jnp.float32),
        ],
    )(x, degout, preW, preb, fc1W, fc1b, fc2W, fc2b, gcnWA, gcnWB, denseWt,
      denseb, f1Wt, f1b)


def _k_layer(acc8, g8, dinv8, based8, biasA, biasB, bdDenseA, bdDenseB,
             bdGcnA, bdGcnB):
    return pl.pallas_call(
        _layer_body,
        grid=(PGRID,),
        in_specs=[
            _phalf_spec, _phalf_spec, _prow_spec(128), _prow_spec(256),
            _wspec((1, 128)), _wspec((1, 128)),
            _wspec((128, 256)), _wspec((128, 256)),
            _wspec((256, 128)), _wspec((256, 128)),
        ],
        out_specs=_phalf_spec,
        out_shape=jax.ShapeDtypeStruct((NC, PR, 128), jnp.float32),
    )(acc8, g8, dinv8, based8, biasA, biasB, bdDenseA, bdDenseB, bdGcnA,
      bdGcnB)


def _k_final(acc8, g8, dinv8, based8, basef8, biasA, biasB, bdDenseA,
             bdDenseB, bdF1, bdF2, f2bt):
    return pl.pallas_call(
        _final_body,
        grid=(PGRID,),
        in_specs=[
            _phalf_spec, _phalf_spec, _prow_spec(128), _prow_spec(256),
            _prow_spec(256),
            _wspec((1, 128)), _wspec((1, 128)),
            _wspec((128, 256)), _wspec((128, 256)),
            _wspec((256, 256)), _wspec((256, 16)), _wspec((1, 16)),
        ],
        out_specs=_prow_spec(16),
        out_shape=jax.ShapeDtypeStruct((PR, 16), jnp.float32),
    )(acc8, g8, dinv8, based8, basef8, biasA, biasB, bdDenseA, bdDenseB,
      bdF1, bdF2, f2bt)


def _block_diag8(W):
    a, b = W.shape
    out = jnp.zeros((8 * a, 8 * b), jnp.float32)
    for j in range(8):
        out = out.at[j * a:(j + 1) * a, j * b:(j + 1) * b].set(W)
    return out


def kernel(x, edge_index, pre_W, pre_b, fc1_W, fc1_b, fc2_W, fc2_b,
           gcn_W, gcn_b, dense_W, dense_b, f1_W, f1_b, f2_W, f2_b):
    src = edge_index[0]
    dst = edge_index[1]
    pad = EPAD - E
    src3 = jnp.concatenate([src, jnp.zeros((pad,), jnp.int32)]).reshape(CROWS, 128)
    dst3 = jnp.concatenate([dst, jnp.full((pad,), DUMMY, jnp.int32)]).reshape(CROWS, 128)
    src2 = jnp.stack([src3, src3 + ACCR])

    degout = _sc_degree(dst3)

    gcnWA = gcn_W[:, 0:HALF]
    gcnWB = gcn_W[:, HALF:D]
    denseWt = dense_W[0:D]
    f1Wt = f1_W[0:D]

    biasA = jnp.tile(gcn_b[0:HALF], 8).reshape(1, 128)
    biasB = jnp.tile(gcn_b[HALF:D], 8).reshape(1, 128)
    bdDenseA = _block_diag8(dense_W[D:D + HALF])
    bdDenseB = _block_diag8(dense_W[D + HALF:2 * D])
    bdGcnA = _block_diag8(gcnWA)
    bdGcnB = _block_diag8(gcnWB)
    bdF1 = _block_diag8(f1_W[D:2 * D])
    bdF2 = _block_diag8(f2_W)
    f2bt = jnp.tile(f2_b, 8).reshape(1, 16)

    dinvw, based, basef, g2 = _k_pre(
        x, degout, pre_W, pre_b.reshape(1, -1), fc1_W, fc1_b.reshape(1, -1),
        fc2_W, fc2_b.reshape(1, -1), gcnWA, gcnWB, denseWt,
        dense_b.reshape(1, -1), f1Wt, f1_b.reshape(1, -1))

    dinv8 = dinvw.reshape(PR, 128)
    based8 = based.reshape(PR, 256)
    basef8 = basef.reshape(PR, 256)
    g8 = g2.reshape(NC, PR, 128)

    for layer in range(6):
        acc2 = _sc_scatter(src2, dst3, g8.reshape(NC * ACCR, HALF))
        acc8 = acc2.reshape(NC, PR, 128)
        if layer < 5:
            g8 = _k_layer(acc8, g8, dinv8, based8, biasA, biasB, bdDenseA,
                          bdDenseB, bdGcnA, bdGcnB)
        else:
            out8 = _k_final(acc8, g8, dinv8, based8, basef8, biasA, biasB,
                            bdDenseA, bdDenseB, bdF1, bdF2, f2bt)
            out = out8.reshape(ACCR, 2)[0:N]
    return out


# speedup vs baseline: 36.5828x; 1.2789x over previous
import functools

import jax
import jax.numpy as jnp
from jax import lax
from jax.experimental import pallas as pl
from jax.experimental.pallas import tpu as pltpu
from jax.experimental.pallas import tpu_sc as plsc

N = 100000
E = 1600000
D = 32
HALF = 16
NC = 2
NS = 16

BLK = 3136
GRID = 32
ACCR = BLK * GRID
DUMMY = N

G_EDGES = 1024
EPAD = 1605632
CROWS = EPAD // 128
ROWS_PER_TILE = CROWS // NS
SUPER = 16
NSUPER = ROWS_PER_TILE // SUPER
ROWS_PER_TILE_DEG = CROWS // (NS * NC)
GROUPS_DEG = ROWS_PER_TILE_DEG // 8
OUT_SLICE = ACCR // NS
ZROWS = OUT_SLICE // 32

_mesh = plsc.VectorSubcoreMesh(core_axis_name="c", subcore_axis_name="s")
_sc_params = pltpu.CompilerParams(use_tc_tiling_on_sc=False)


def _fill(ref, rows, value):
    v = jnp.full((16,), value, jnp.float32)

    def body(i, _):
        ref[i, :] = v
        return 0

    lax.fori_loop(0, rows, body, 0)


def _fill3(ref, d0, d1, value):
    v = jnp.full((16,), value, jnp.float32)

    def body(i, _):
        ref[i // d1, i % d1, :] = v
        return 0

    lax.fori_loop(0, d0 * d1, body, 0)


@functools.partial(
    pl.kernel,
    out_type=jax.ShapeDtypeStruct((NC, ACCR, HALF), jnp.float32),
    mesh=_mesh,
    scratch_types=dict(
        didx=pltpu.VMEM((8, 128), jnp.int32),
        ones=pltpu.VMEM((8, 128, HALF), jnp.float32),
        stage=pltpu.VMEM((ZROWS, HALF), jnp.float32),
        acc=pltpu.VMEM_SHARED((ACCR, HALF), jnp.float32),
    ),
    compiler_params=_sc_params,
)
def _sc_degree(dst_hbm, out_hbm, didx, ones, stage, acc):
    c = lax.axis_index("c")
    s = lax.axis_index("s")
    w = s * NC + c

    _fill(stage, ZROWS, 0.0)
    _fill3(ones, 8, 128, 1.0)
    for k in range(32):
        pltpu.sync_copy(stage, acc.at[pl.ds(s * OUT_SLICE + k * ZROWS, ZROWS)])
    plsc.subcore_barrier()

    def body(g, _):
        base = w * ROWS_PER_TILE_DEG + g * 8
        pltpu.sync_copy(dst_hbm.at[pl.ds(base, 8), :], didx)
        for j in range(8):
            pltpu.sync_copy(ones.at[j], acc.at[didx.at[j]], add=True)
        return 0

    lax.fori_loop(0, GROUPS_DEG, body, 0)
    plsc.subcore_barrier()

    for k in range(32):
        off = s * OUT_SLICE + k * ZROWS
        pltpu.sync_copy(acc.at[pl.ds(off, ZROWS)], stage)
        pltpu.sync_copy(stage, out_hbm.at[c, pl.ds(off, ZROWS), :])


@functools.partial(
    pl.kernel,
    out_type=jax.ShapeDtypeStruct((NC, ACCR, HALF), jnp.float32),
    mesh=_mesh,
    scratch_types=dict(
        ebuf=pltpu.VMEM((2, SUPER, 2, 128), jnp.int32),
        rows=pltpu.VMEM((8, 128, HALF), jnp.float32),
        stage=pltpu.VMEM((ZROWS, HALF), jnp.float32),
        acc=pltpu.VMEM_SHARED((ACCR, HALF), jnp.float32),
        isem=pltpu.SemaphoreType.DMA((2,)),
        gsem=pltpu.SemaphoreType.DMA((8,)),
        ssem=pltpu.SemaphoreType.DMA((8,)),
        zsem=pltpu.SemaphoreType.DMA,
    ),
    compiler_params=_sc_params,
)
def _sc_scatter(edges_hbm, g_hbm, out_hbm, ebuf, rows, stage, acc,
                isem, gsem, ssem, zsem):
    c = lax.axis_index("c")
    s = lax.axis_index("s")
    tbase = s * ROWS_PER_TILE

    _fill(stage, ZROWS, 0.0)
    for k in range(32):
        pltpu.async_copy(stage, acc.at[pl.ds(s * OUT_SLICE + k * ZROWS, ZROWS)],
                         zsem)
    for k in range(32):
        pltpu.make_async_copy(
            stage, acc.at[pl.ds(s * OUT_SLICE + k * ZROWS, ZROWS)], zsem).wait()
    plsc.subcore_barrier()

    def _wait_scatter(j):
        pltpu.make_async_copy(rows.at[j], acc.at[pl.ds(0, 128)], ssem.at[j]).wait()

    pltpu.async_copy(edges_hbm.at[c, pl.ds(tbase, SUPER), :, :], ebuf.at[0],
                     isem.at[0])

    def body(S, _):
        par = lax.rem(S, 2)
        pltpu.make_async_copy(edges_hbm.at[c, pl.ds(0, SUPER), :, :],
                              ebuf.at[par], isem.at[par]).wait()
        for h in range(2):
            if h == 0:
                @pl.when(S > 0)
                def _():
                    for j in range(8):
                        _wait_scatter(j)

                @pl.when(S < NSUPER - 1)
                def _():
                    pltpu.async_copy(
                        edges_hbm.at[c, pl.ds(tbase + (S + 1) * SUPER, SUPER),
                                     :, :],
                        ebuf.at[1 - par], isem.at[1 - par])
            else:
                for j in range(8):
                    _wait_scatter(j)
            for j in range(8):
                pltpu.async_copy(g_hbm.at[ebuf.at[par, h * 8 + j, 0]],
                                 rows.at[j], gsem.at[j])
            for j in range(8):
                pltpu.make_async_copy(g_hbm.at[pl.ds(0, 128)], rows.at[j],
                                      gsem.at[j]).wait()
                pltpu.async_copy(rows.at[j], acc.at[ebuf.at[par, h * 8 + j, 1]],
                                 ssem.at[j], add=True)
        return 0

    lax.fori_loop(0, NSUPER, body, 0)
    for j in range(8):
        _wait_scatter(j)
    plsc.subcore_barrier()

    for k in range(32):
        off = s * OUT_SLICE + k * ZROWS
        pltpu.sync_copy(acc.at[pl.ds(off, ZROWS)], stage)
        pltpu.sync_copy(stage, out_hbm.at[c, pl.ds(off, ZROWS), :])


def _dot(a, b):
    return jax.lax.dot_general(a, b, (((1,), (0,)), ((), ())),
                               preferred_element_type=jnp.float32)


def _wspec(shape):
    return pl.BlockSpec(shape, lambda i: tuple(0 for _ in shape))


def _pre_body(x_ref, deg_ref, bdPre, prebT, bdFc1, fc1bT, bdFc2, fc2bT,
              bdGcnA, bdGcnB, bdDenseT, densebT, bdF1T, f1bT,
              dinv_ref, based_ref, basef_ref, g2_ref):
    x1 = jax.nn.sigmoid(_dot(x_ref[...], bdPre[...]) + prebT[...])
    ni = jax.nn.relu(_dot(x1, bdFc1[...]) + fc1bT[...])
    h0 = jax.nn.relu(_dot(x1, bdFc2[...]) + fc2bT[...])
    db = deg_ref[...]
    dinv = lax.rsqrt(db[0] + db[1] + 1.0)
    dinv_ref[...] = dinv
    based_ref[...] = _dot(ni, bdDenseT[...]) + densebT[...]
    basef_ref[...] = _dot(ni, bdF1T[...]) + f1bT[...]
    g2_ref[0] = _dot(h0, bdGcnA[...]) * dinv
    g2_ref[1] = _dot(h0, bdGcnB[...]) * dinv


def _layer_body(acc_ref, g_ref, dinv_ref, based_ref, biasA, biasB,
                bdDenseA, bdDenseB, bdGcnA, bdGcnB, gout_ref):
    a = acc_ref[...]
    gb = g_ref[...]
    dinv = dinv_ref[...]
    goA = jax.nn.relu(dinv * (a[0] + gb[0]) + biasA[...])
    goB = jax.nn.relu(dinv * (a[1] + gb[1]) + biasB[...])
    h = jax.nn.relu(_dot(goA, bdDenseA[...]) + _dot(goB, bdDenseB[...])
                    + based_ref[...])
    gout_ref[0] = _dot(h, bdGcnA[...]) * dinv
    gout_ref[1] = _dot(h, bdGcnB[...]) * dinv


def _final_body(acc_ref, g_ref, dinv_ref, based_ref, basef_ref, biasA, biasB,
                bdDenseA, bdDenseB, bdF1, bdF2, f2bt, out_ref):
    a = acc_ref[...]
    gb = g_ref[...]
    dinv = dinv_ref[...]
    goA = jax.nn.relu(dinv * (a[0] + gb[0]) + biasA[...])
    goB = jax.nn.relu(dinv * (a[1] + gb[1]) + biasB[...])
    h = jax.nn.relu(_dot(goA, bdDenseA[...]) + _dot(goB, bdDenseB[...])
                    + based_ref[...])
    h2 = jax.nn.relu(_dot(h, bdF1[...]) + basef_ref[...])
    out_ref[...] = _dot(h2, bdF2[...]) + f2bt[...]


PR = ACCR // 8
PBLK = BLK // 8
PGRID = GRID

_phalf_spec = pl.BlockSpec((NC, PBLK, 128), lambda i: (0, i, 0))
_prow_spec = lambda w: pl.BlockSpec((PBLK, w), lambda i: (i, 0))
_half_spec = pl.BlockSpec((NC, BLK, HALF), lambda i: (0, i, 0))
_row_spec = lambda w: pl.BlockSpec((BLK, w), lambda i: (i, 0))


def _k_pre(x8, deg8, bdPre, prebT, bdFc1, fc1bT, bdFc2, fc2bT, bdGcnA,
           bdGcnB, bdDenseT, densebT, bdF1T, f1bT):
    return pl.pallas_call(
        _pre_body,
        grid=(GRID,),
        in_specs=[
            _prow_spec(48), _phalf_spec,
            _wspec((48, 80)), _wspec((1, 80)),
            _wspec((80, 256)), _wspec((1, 256)),
            _wspec((80, 256)), _wspec((1, 256)),
            _wspec((256, 128)), _wspec((256, 128)),
            _wspec((256, 256)), _wspec((1, 256)),
            _wspec((256, 256)), _wspec((1, 256)),
        ],
        out_specs=[_prow_spec(128), _prow_spec(256), _prow_spec(256),
                   _phalf_spec],
        out_shape=[
            jax.ShapeDtypeStruct((PR, 128), jnp.float32),
            jax.ShapeDtypeStruct((PR, 256), jnp.float32),
            jax.ShapeDtypeStruct((PR, 256), jnp.float32),
            jax.ShapeDtypeStruct((NC, PR, 128), jnp.float32),
        ],
    )(x8, deg8, bdPre, prebT, bdFc1, fc1bT, bdFc2, fc2bT, bdGcnA, bdGcnB,
      bdDenseT, densebT, bdF1T, f1bT)


def _k_layer(acc8, g8, dinv8, based8, biasA, biasB, bdDenseA, bdDenseB,
             bdGcnA, bdGcnB):
    return pl.pallas_call(
        _layer_body,
        grid=(PGRID,),
        in_specs=[
            _phalf_spec, _phalf_spec, _prow_spec(128), _prow_spec(256),
            _wspec((1, 128)), _wspec((1, 128)),
            _wspec((128, 256)), _wspec((128, 256)),
            _wspec((256, 128)), _wspec((256, 128)),
        ],
        out_specs=_phalf_spec,
        out_shape=jax.ShapeDtypeStruct((NC, PR, 128), jnp.float32),
    )(acc8, g8, dinv8, based8, biasA, biasB, bdDenseA, bdDenseB, bdGcnA,
      bdGcnB)


def _k_final(acc8, g8, dinv8, based8, basef8, biasA, biasB, bdDenseA,
             bdDenseB, bdF1, bdF2, f2bt):
    return pl.pallas_call(
        _final_body,
        grid=(PGRID,),
        in_specs=[
            _phalf_spec, _phalf_spec, _prow_spec(128), _prow_spec(256),
            _prow_spec(256),
            _wspec((1, 128)), _wspec((1, 128)),
            _wspec((128, 256)), _wspec((128, 256)),
            _wspec((256, 256)), _wspec((256, 16)), _wspec((1, 16)),
        ],
        out_specs=_prow_spec(16),
        out_shape=jax.ShapeDtypeStruct((PR, 16), jnp.float32),
    )(acc8, g8, dinv8, based8, basef8, biasA, biasB, bdDenseA, bdDenseB,
      bdF1, bdF2, f2bt)


def _block_diag8(W):
    a, b = W.shape
    out = jnp.zeros((8 * a, 8 * b), jnp.float32)
    for j in range(8):
        out = out.at[j * a:(j + 1) * a, j * b:(j + 1) * b].set(W)
    return out


def kernel(x, edge_index, pre_W, pre_b, fc1_W, fc1_b, fc2_W, fc2_b,
           gcn_W, gcn_b, dense_W, dense_b, f1_W, f1_b, f2_W, f2_b):
    src = edge_index[0]
    dst = edge_index[1]
    pad = EPAD - E
    src3 = jnp.concatenate([src, jnp.zeros((pad,), jnp.int32)]).reshape(CROWS, 128)
    dst3 = jnp.concatenate([dst, jnp.full((pad,), DUMMY, jnp.int32)]).reshape(CROWS, 128)
    edges = jnp.stack([
        jnp.stack([src3, dst3], axis=1),
        jnp.stack([src3 + ACCR, dst3], axis=1),
    ])

    degout = _sc_degree(dst3)
    deg8 = degout.reshape(NC, PR, 128)

    biasA = jnp.tile(gcn_b[0:HALF], 8).reshape(1, 128)
    biasB = jnp.tile(gcn_b[HALF:D], 8).reshape(1, 128)
    bdDenseA = _block_diag8(dense_W[D:D + HALF])
    bdDenseB = _block_diag8(dense_W[D + HALF:2 * D])
    bdGcnA = _block_diag8(gcn_W[:, 0:HALF])
    bdGcnB = _block_diag8(gcn_W[:, HALF:D])
    bdF1 = _block_diag8(f1_W[D:2 * D])
    bdF2 = _block_diag8(f2_W)
    f2bt = jnp.tile(f2_b, 8).reshape(1, 16)

    bdPre = _block_diag8(pre_W)
    bdFc1 = _block_diag8(fc1_W)
    bdFc2 = _block_diag8(fc2_W)
    bdDenseT = _block_diag8(dense_W[0:D])
    bdF1T = _block_diag8(f1_W[0:D])

    x8 = jnp.pad(x, ((0, ACCR - N), (0, 0))).reshape(PR, 48)

    dinv8, based8, basef8, g8 = _k_pre(
        x8, deg8, bdPre, jnp.tile(pre_b, 8).reshape(1, -1),
        bdFc1, jnp.tile(fc1_b, 8).reshape(1, -1),
        bdFc2, jnp.tile(fc2_b, 8).reshape(1, -1),
        bdGcnA, bdGcnB, bdDenseT, jnp.tile(dense_b, 8).reshape(1, -1),
        bdF1T, jnp.tile(f1_b, 8).reshape(1, -1))

    for layer in range(6):
        acc2 = _sc_scatter(edges, g8.reshape(NC * ACCR, HALF))
        acc8 = acc2.reshape(NC, PR, 128)
        if layer < 5:
            g8 = _k_layer(acc8, g8, dinv8, based8, biasA, biasB, bdDenseA,
                          bdDenseB, bdGcnA, bdGcnB)
        else:
            out8 = _k_final(acc8, g8, dinv8, based8, basef8, biasA, biasB,
                            bdDenseA, bdDenseB, bdF1, bdF2, f2bt)
            out = out8.reshape(ACCR, 2)[0:N]
    return out


# speedup vs baseline: 38.6576x; 1.0567x over previous
import functools

import jax
import jax.numpy as jnp
from jax import lax
from jax.experimental import pallas as pl
from jax.experimental.pallas import tpu as pltpu
from jax.experimental.pallas import tpu_sc as plsc

N = 100000
E = 1600000
D = 32
HALF = 16
NC = 2
NS = 16

BLK = 3136
GRID = 32
ACCR = BLK * GRID
DUMMY = N

G_EDGES = 1024
EPAD = 1605632
CROWS = EPAD // 128
ROWS_PER_TILE = CROWS // NS
SUPER = 16
NSUPER = ROWS_PER_TILE // SUPER
ROWS_PER_TILE_DEG = CROWS // (NS * NC)
GROUPS_DEG = ROWS_PER_TILE_DEG // 8
OUT_SLICE = ACCR // NS
ZROWS = OUT_SLICE // 32

_mesh = plsc.VectorSubcoreMesh(core_axis_name="c", subcore_axis_name="s")
_sc_params = pltpu.CompilerParams(use_tc_tiling_on_sc=False)


def _fill(ref, rows, value):
    v = jnp.full((16,), value, jnp.float32)

    def body(i, _):
        ref[i, :] = v
        return 0

    lax.fori_loop(0, rows, body, 0)


def _fill3(ref, d0, d1, value):
    v = jnp.full((16,), value, jnp.float32)

    def body(i, _):
        ref[i // d1, i % d1, :] = v
        return 0

    lax.fori_loop(0, d0 * d1, body, 0)


@functools.partial(
    pl.kernel,
    out_type=jax.ShapeDtypeStruct((NC, ACCR, HALF), jnp.float32),
    mesh=_mesh,
    scratch_types=dict(
        dbuf=pltpu.VMEM((2, 8, 128), jnp.int32),
        ones=pltpu.VMEM((128, HALF), jnp.float32),
        stage=pltpu.VMEM((ZROWS, HALF), jnp.float32),
        acc=pltpu.VMEM_SHARED((ACCR, HALF), jnp.float32),
        isem=pltpu.SemaphoreType.DMA((2,)),
        ssem=pltpu.SemaphoreType.DMA,
        zsem=pltpu.SemaphoreType.DMA,
    ),
    compiler_params=_sc_params,
)
def _sc_degree(dst_hbm, out_hbm, dbuf, ones, stage, acc, isem, ssem, zsem):
    c = lax.axis_index("c")
    s = lax.axis_index("s")
    w = s * NC + c
    tbase = w * ROWS_PER_TILE_DEG

    _fill(stage, ZROWS, 0.0)
    _fill(ones, 128, 1.0)
    for k in range(32):
        pltpu.async_copy(stage, acc.at[pl.ds(s * OUT_SLICE + k * ZROWS, ZROWS)],
                         zsem)
    for k in range(32):
        pltpu.make_async_copy(
            stage, acc.at[pl.ds(s * OUT_SLICE + k * ZROWS, ZROWS)], zsem).wait()
    plsc.subcore_barrier()

    def _wait_scatter():
        pltpu.make_async_copy(ones, acc.at[pl.ds(0, 128)], ssem).wait()

    pltpu.async_copy(dst_hbm.at[pl.ds(tbase, 8), :], dbuf.at[0], isem.at[0])

    def body(S, _):
        par = lax.rem(S, 2)
        pltpu.make_async_copy(dst_hbm.at[pl.ds(0, 8), :], dbuf.at[par],
                              isem.at[par]).wait()

        @pl.when(S > 0)
        def _():
            for r in range(8):
                _wait_scatter()

        @pl.when(S < GROUPS_DEG - 1)
        def _():
            pltpu.async_copy(dst_hbm.at[pl.ds(tbase + (S + 1) * 8, 8), :],
                             dbuf.at[1 - par], isem.at[1 - par])

        for r in range(8):
            pltpu.async_copy(ones, acc.at[dbuf.at[par, r]], ssem, add=True)
        return 0

    lax.fori_loop(0, GROUPS_DEG, body, 0)
    for r in range(8):
        _wait_scatter()
    plsc.subcore_barrier()

    for k in range(32):
        off = s * OUT_SLICE + k * ZROWS
        pltpu.sync_copy(acc.at[pl.ds(off, ZROWS)], stage)
        pltpu.sync_copy(stage, out_hbm.at[c, pl.ds(off, ZROWS), :])


@functools.partial(
    pl.kernel,
    out_type=jax.ShapeDtypeStruct((NC, ACCR, HALF), jnp.float32),
    mesh=_mesh,
    scratch_types=dict(
        ebuf=pltpu.VMEM((2, SUPER, 2, 128), jnp.int32),
        rows=pltpu.VMEM((8, 128, HALF), jnp.float32),
        stage=pltpu.VMEM((ZROWS, HALF), jnp.float32),
        acc=pltpu.VMEM_SHARED((ACCR, HALF), jnp.float32),
        isem=pltpu.SemaphoreType.DMA((2,)),
        gsem=pltpu.SemaphoreType.DMA((8,)),
        ssem=pltpu.SemaphoreType.DMA((8,)),
        zsem=pltpu.SemaphoreType.DMA,
    ),
    compiler_params=_sc_params,
)
def _sc_scatter(edges_hbm, g_hbm, out_hbm, ebuf, rows, stage, acc,
                isem, gsem, ssem, zsem):
    c = lax.axis_index("c")
    s = lax.axis_index("s")
    tbase = s * ROWS_PER_TILE

    _fill(stage, ZROWS, 0.0)
    for k in range(32):
        pltpu.async_copy(stage, acc.at[pl.ds(s * OUT_SLICE + k * ZROWS, ZROWS)],
                         zsem)
    for k in range(32):
        pltpu.make_async_copy(
            stage, acc.at[pl.ds(s * OUT_SLICE + k * ZROWS, ZROWS)], zsem).wait()
    plsc.subcore_barrier()

    def _wait_scatter(j):
        pltpu.make_async_copy(rows.at[j], acc.at[pl.ds(0, 128)], ssem.at[j]).wait()

    pltpu.async_copy(edges_hbm.at[pl.ds(tbase, SUPER), :, :], ebuf.at[0],
                     isem.at[0])

    def body(S, _):
        par = lax.rem(S, 2)
        pltpu.make_async_copy(edges_hbm.at[pl.ds(0, SUPER), :, :],
                              ebuf.at[par], isem.at[par]).wait()
        for h in range(2):
            if h == 0:
                @pl.when(S > 0)
                def _():
                    for j in range(8):
                        _wait_scatter(j)

                @pl.when(S < NSUPER - 1)
                def _():
                    pltpu.async_copy(
                        edges_hbm.at[pl.ds(tbase + (S + 1) * SUPER, SUPER),
                                     :, :],
                        ebuf.at[1 - par], isem.at[1 - par])
            else:
                for j in range(8):
                    _wait_scatter(j)
            for j in range(8):
                pltpu.async_copy(g_hbm.at[c].at[ebuf.at[par, h * 8 + j, 0]],
                                 rows.at[j], gsem.at[j])
            for j in range(8):
                pltpu.make_async_copy(g_hbm.at[c].at[pl.ds(0, 128)], rows.at[j],
                                      gsem.at[j]).wait()
                pltpu.async_copy(rows.at[j], acc.at[ebuf.at[par, h * 8 + j, 1]],
                                 ssem.at[j], add=True)
        return 0

    lax.fori_loop(0, NSUPER, body, 0)
    for j in range(8):
        _wait_scatter(j)
    plsc.subcore_barrier()

    for k in range(32):
        off = s * OUT_SLICE + k * ZROWS
        pltpu.sync_copy(acc.at[pl.ds(off, ZROWS)], stage)
        pltpu.sync_copy(stage, out_hbm.at[c, pl.ds(off, ZROWS), :])


def _dot(a, b):
    return jax.lax.dot_general(a, b, (((1,), (0,)), ((), ())),
                               preferred_element_type=jnp.float32)


def _wspec(shape):
    return pl.BlockSpec(shape, lambda i: tuple(0 for _ in shape))


def _pre_body(x_ref, deg_ref, bdPre, prebT, bdFc1, fc1bT, bdFc2, fc2bT,
              bdGcnA, bdGcnB, bdDenseT, densebT, bdF1T, f1bT,
              dinv_ref, based_ref, basef_ref, g2_ref):
    x1 = jax.nn.sigmoid(_dot(x_ref[...], bdPre[...]) + prebT[...])
    ni = jax.nn.relu(_dot(x1, bdFc1[...]) + fc1bT[...])
    h0 = jax.nn.relu(_dot(x1, bdFc2[...]) + fc2bT[...])
    db = deg_ref[...]
    dinv = lax.rsqrt(db[0] + db[1] + 1.0)
    dinv_ref[...] = dinv
    based_ref[...] = _dot(ni, bdDenseT[...]) + densebT[...]
    basef_ref[...] = _dot(ni, bdF1T[...]) + f1bT[...]
    g2_ref[0] = _dot(h0, bdGcnA[...]) * dinv
    g2_ref[1] = _dot(h0, bdGcnB[...]) * dinv


def _layer_body(acc_ref, g_ref, dinv_ref, based_ref, biasA, biasB,
                bdDenseA, bdDenseB, bdGcnA, bdGcnB, gout_ref):
    a = acc_ref[...]
    gb = g_ref[...]
    dinv = dinv_ref[...]
    goA = jax.nn.relu(dinv * (a[0] + gb[0]) + biasA[...])
    goB = jax.nn.relu(dinv * (a[1] + gb[1]) + biasB[...])
    h = jax.nn.relu(_dot(goA, bdDenseA[...]) + _dot(goB, bdDenseB[...])
                    + based_ref[...])
    gout_ref[0] = _dot(h, bdGcnA[...]) * dinv
    gout_ref[1] = _dot(h, bdGcnB[...]) * dinv


def _final_body(acc_ref, g_ref, dinv_ref, based_ref, basef_ref, biasA, biasB,
                bdDenseA, bdDenseB, bdF1, bdF2, f2bt, out_ref):
    a = acc_ref[...]
    gb = g_ref[...]
    dinv = dinv_ref[...]
    goA = jax.nn.relu(dinv * (a[0] + gb[0]) + biasA[...])
    goB = jax.nn.relu(dinv * (a[1] + gb[1]) + biasB[...])
    h = jax.nn.relu(_dot(goA, bdDenseA[...]) + _dot(goB, bdDenseB[...])
                    + based_ref[...])
    h2 = jax.nn.relu(_dot(h, bdF1[...]) + basef_ref[...])
    out_ref[...] = _dot(h2, bdF2[...]) + f2bt[...]


PR = ACCR // 8
PBLK = BLK // 8
PGRID = GRID

_phalf_spec = pl.BlockSpec((NC, PBLK, 128), lambda i: (0, i, 0))
_prow_spec = lambda w: pl.BlockSpec((PBLK, w), lambda i: (i, 0))
_half_spec = pl.BlockSpec((NC, BLK, HALF), lambda i: (0, i, 0))
_row_spec = lambda w: pl.BlockSpec((BLK, w), lambda i: (i, 0))


def _k_pre(x8, deg8, bdPre, prebT, bdFc1, fc1bT, bdFc2, fc2bT, bdGcnA,
           bdGcnB, bdDenseT, densebT, bdF1T, f1bT):
    return pl.pallas_call(
        _pre_body,
        grid=(GRID,),
        in_specs=[
            _prow_spec(48), _phalf_spec,
            _wspec((48, 80)), _wspec((1, 80)),
            _wspec((80, 256)), _wspec((1, 256)),
            _wspec((80, 256)), _wspec((1, 256)),
            _wspec((256, 128)), _wspec((256, 128)),
            _wspec((256, 256)), _wspec((1, 256)),
            _wspec((256, 256)), _wspec((1, 256)),
        ],
        out_specs=[_prow_spec(128), _prow_spec(256), _prow_spec(256),
                   _phalf_spec],
        out_shape=[
            jax.ShapeDtypeStruct((PR, 128), jnp.float32),
            jax.ShapeDtypeStruct((PR, 256), jnp.float32),
            jax.ShapeDtypeStruct((PR, 256), jnp.float32),
            jax.ShapeDtypeStruct((NC, PR, 128), jnp.float32),
        ],
    )(x8, deg8, bdPre, prebT, bdFc1, fc1bT, bdFc2, fc2bT, bdGcnA, bdGcnB,
      bdDenseT, densebT, bdF1T, f1bT)


def _k_layer(acc8, g8, dinv8, based8, biasA, biasB, bdDenseA, bdDenseB,
             bdGcnA, bdGcnB):
    return pl.pallas_call(
        _layer_body,
        grid=(PGRID,),
        in_specs=[
            _phalf_spec, _phalf_spec, _prow_spec(128), _prow_spec(256),
            _wspec((1, 128)), _wspec((1, 128)),
            _wspec((128, 256)), _wspec((128, 256)),
            _wspec((256, 128)), _wspec((256, 128)),
        ],
        out_specs=_phalf_spec,
        out_shape=jax.ShapeDtypeStruct((NC, PR, 128), jnp.float32),
    )(acc8, g8, dinv8, based8, biasA, biasB, bdDenseA, bdDenseB, bdGcnA,
      bdGcnB)


def _k_final(acc8, g8, dinv8, based8, basef8, biasA, biasB, bdDenseA,
             bdDenseB, bdF1, bdF2, f2bt):
    return pl.pallas_call(
        _final_body,
        grid=(PGRID,),
        in_specs=[
            _phalf_spec, _phalf_spec, _prow_spec(128), _prow_spec(256),
            _prow_spec(256),
            _wspec((1, 128)), _wspec((1, 128)),
            _wspec((128, 256)), _wspec((128, 256)),
            _wspec((256, 256)), _wspec((256, 16)), _wspec((1, 16)),
        ],
        out_specs=_prow_spec(16),
        out_shape=jax.ShapeDtypeStruct((PR, 16), jnp.float32),
    )(acc8, g8, dinv8, based8, basef8, biasA, biasB, bdDenseA, bdDenseB,
      bdF1, bdF2, f2bt)


def _block_diag8(W):
    a, b = W.shape
    out = jnp.zeros((8 * a, 8 * b), jnp.float32)
    for j in range(8):
        out = out.at[j * a:(j + 1) * a, j * b:(j + 1) * b].set(W)
    return out


def kernel(x, edge_index, pre_W, pre_b, fc1_W, fc1_b, fc2_W, fc2_b,
           gcn_W, gcn_b, dense_W, dense_b, f1_W, f1_b, f2_W, f2_b):
    src = edge_index[0]
    dst = edge_index[1]
    pad = EPAD - E
    src3 = jnp.concatenate([src, jnp.zeros((pad,), jnp.int32)]).reshape(CROWS, 128)
    dst3 = jnp.concatenate([dst, jnp.full((pad,), DUMMY, jnp.int32)]).reshape(CROWS, 128)
    edges = jnp.stack([src3, dst3], axis=1)

    degout = _sc_degree(dst3)
    deg8 = degout.reshape(NC, PR, 128)

    biasA = jnp.tile(gcn_b[0:HALF], 8).reshape(1, 128)
    biasB = jnp.tile(gcn_b[HALF:D], 8).reshape(1, 128)
    bdDenseA = _block_diag8(dense_W[D:D + HALF])
    bdDenseB = _block_diag8(dense_W[D + HALF:2 * D])
    bdGcnA = _block_diag8(gcn_W[:, 0:HALF])
    bdGcnB = _block_diag8(gcn_W[:, HALF:D])
    bdF1 = _block_diag8(f1_W[D:2 * D])
    bdF2 = _block_diag8(f2_W)
    f2bt = jnp.tile(f2_b, 8).reshape(1, 16)

    bdPre = _block_diag8(pre_W)
    bdFc1 = _block_diag8(fc1_W)
    bdFc2 = _block_diag8(fc2_W)
    bdDenseT = _block_diag8(dense_W[0:D])
    bdF1T = _block_diag8(f1_W[0:D])

    x8 = x.reshape(N // 8, 48)

    dinv8, based8, basef8, g8 = _k_pre(
        x8, deg8, bdPre, jnp.tile(pre_b, 8).reshape(1, -1),
        bdFc1, jnp.tile(fc1_b, 8).reshape(1, -1),
        bdFc2, jnp.tile(fc2_b, 8).reshape(1, -1),
        bdGcnA, bdGcnB, bdDenseT, jnp.tile(dense_b, 8).reshape(1, -1),
        bdF1T, jnp.tile(f1_b, 8).reshape(1, -1))

    for layer in range(6):
        acc2 = _sc_scatter(edges, g8.reshape(NC, ACCR, HALF))
        acc8 = acc2.reshape(NC, PR, 128)
        if layer < 5:
            g8 = _k_layer(acc8, g8, dinv8, based8, biasA, biasB, bdDenseA,
                          bdDenseB, bdGcnA, bdGcnB)
        else:
            out8 = _k_final(acc8, g8, dinv8, based8, basef8, biasA, biasB,
                            bdDenseA, bdDenseB, bdF1, bdF2, f2bt)
            out = out8.reshape(ACCR, 2)[0:N]
    return out


# speedup vs baseline: 40.8966x; 1.0579x over previous
import functools

import jax
import jax.numpy as jnp
from jax import lax
from jax.experimental import pallas as pl
from jax.experimental.pallas import tpu as pltpu
from jax.experimental.pallas import tpu_sc as plsc

N = 100000
E = 1600000
D = 32
HALF = 16
NC = 2
NS = 16

BLK = 6272
GRID = 16
ACCR = BLK * GRID
DUMMY = N

G_EDGES = 1024
EPAD = 1605632
CROWS = EPAD // 128
ROWS_PER_TILE = CROWS // NS
SUPER = 16
NSUPER = ROWS_PER_TILE // SUPER
ROWS_PER_TILE_DEG = CROWS // (NS * NC)
GROUPS_DEG = ROWS_PER_TILE_DEG // 8
OUT_SLICE = ACCR // NS
ZROWS = OUT_SLICE // 32

_mesh = plsc.VectorSubcoreMesh(core_axis_name="c", subcore_axis_name="s")
_sc_params = pltpu.CompilerParams(use_tc_tiling_on_sc=False)


def _fill(ref, rows, value):
    v = jnp.full((16,), value, jnp.float32)

    def body(i, _):
        ref[i, :] = v
        return 0

    lax.fori_loop(0, rows, body, 0)


def _fill3(ref, d0, d1, value):
    v = jnp.full((16,), value, jnp.float32)

    def body(i, _):
        ref[i // d1, i % d1, :] = v
        return 0

    lax.fori_loop(0, d0 * d1, body, 0)


@functools.partial(
    pl.kernel,
    out_type=jax.ShapeDtypeStruct((NC, ACCR, HALF), jnp.float32),
    mesh=_mesh,
    scratch_types=dict(
        dbuf=pltpu.VMEM((2, 8, 128), jnp.int32),
        ones=pltpu.VMEM((128, HALF), jnp.float32),
        stage=pltpu.VMEM((ZROWS, HALF), jnp.float32),
        acc=pltpu.VMEM_SHARED((ACCR, HALF), jnp.float32),
        isem=pltpu.SemaphoreType.DMA((2,)),
        ssem=pltpu.SemaphoreType.DMA,
        zsem=pltpu.SemaphoreType.DMA,
    ),
    compiler_params=_sc_params,
)
def _sc_degree(dst_hbm, out_hbm, dbuf, ones, stage, acc, isem, ssem, zsem):
    c = lax.axis_index("c")
    s = lax.axis_index("s")
    w = s * NC + c
    tbase = w * ROWS_PER_TILE_DEG

    _fill(stage, ZROWS, 0.0)
    _fill(ones, 128, 1.0)
    for k in range(32):
        pltpu.async_copy(stage, acc.at[pl.ds(s * OUT_SLICE + k * ZROWS, ZROWS)],
                         zsem)
    for k in range(32):
        pltpu.make_async_copy(
            stage, acc.at[pl.ds(s * OUT_SLICE + k * ZROWS, ZROWS)], zsem).wait()
    plsc.subcore_barrier()

    def _wait_scatter():
        pltpu.make_async_copy(ones, acc.at[pl.ds(0, 128)], ssem).wait()

    pltpu.async_copy(dst_hbm.at[pl.ds(tbase, 8), :], dbuf.at[0], isem.at[0])

    def body(S, _):
        par = lax.rem(S, 2)
        pltpu.make_async_copy(dst_hbm.at[pl.ds(0, 8), :], dbuf.at[par],
                              isem.at[par]).wait()

        @pl.when(S > 0)
        def _():
            for r in range(8):
                _wait_scatter()

        @pl.when(S < GROUPS_DEG - 1)
        def _():
            pltpu.async_copy(dst_hbm.at[pl.ds(tbase + (S + 1) * 8, 8), :],
                             dbuf.at[1 - par], isem.at[1 - par])

        for r in range(8):
            pltpu.async_copy(ones, acc.at[dbuf.at[par, r]], ssem, add=True)
        return 0

    lax.fori_loop(0, GROUPS_DEG, body, 0)
    for r in range(8):
        _wait_scatter()
    plsc.subcore_barrier()

    for k in range(4):
        off = s * OUT_SLICE + k * (OUT_SLICE // 4)
        pltpu.sync_copy(acc.at[pl.ds(off, OUT_SLICE // 4)],
                        out_hbm.at[c, pl.ds(off, OUT_SLICE // 4), :])


@functools.partial(
    pl.kernel,
    out_type=jax.ShapeDtypeStruct((NC, ACCR, HALF), jnp.float32),
    mesh=_mesh,
    scratch_types=dict(
        ebuf=pltpu.VMEM((2, SUPER, 2, 128), jnp.int32),
        rows=pltpu.VMEM((8, 128, HALF), jnp.float32),
        stage=pltpu.VMEM((ZROWS, HALF), jnp.float32),
        acc=pltpu.VMEM_SHARED((ACCR, HALF), jnp.float32),
        isem=pltpu.SemaphoreType.DMA((2,)),
        gsem=pltpu.SemaphoreType.DMA((8,)),
        ssem=pltpu.SemaphoreType.DMA((8,)),
        zsem=pltpu.SemaphoreType.DMA,
    ),
    compiler_params=_sc_params,
)
def _sc_scatter(edges_hbm, g_hbm, out_hbm, ebuf, rows, stage, acc,
                isem, gsem, ssem, zsem):
    c = lax.axis_index("c")
    s = lax.axis_index("s")
    tbase = s * ROWS_PER_TILE

    _fill(stage, ZROWS, 0.0)
    for k in range(32):
        pltpu.async_copy(stage, acc.at[pl.ds(s * OUT_SLICE + k * ZROWS, ZROWS)],
                         zsem)
    for k in range(32):
        pltpu.make_async_copy(
            stage, acc.at[pl.ds(s * OUT_SLICE + k * ZROWS, ZROWS)], zsem).wait()
    plsc.subcore_barrier()

    def _wait_scatter(j):
        pltpu.make_async_copy(rows.at[j], acc.at[pl.ds(0, 128)], ssem.at[j]).wait()

    pltpu.async_copy(edges_hbm.at[pl.ds(tbase, SUPER), :, :], ebuf.at[0],
                     isem.at[0])

    def body(S, _):
        par = lax.rem(S, 2)
        pltpu.make_async_copy(edges_hbm.at[pl.ds(0, SUPER), :, :],
                              ebuf.at[par], isem.at[par]).wait()
        for h in range(2):
            if h == 0:
                @pl.when(S > 0)
                def _():
                    for j in range(8):
                        _wait_scatter(j)

                @pl.when(S < NSUPER - 1)
                def _():
                    pltpu.async_copy(
                        edges_hbm.at[pl.ds(tbase + (S + 1) * SUPER, SUPER),
                                     :, :],
                        ebuf.at[1 - par], isem.at[1 - par])
            else:
                for j in range(8):
                    _wait_scatter(j)
            for j in range(8):
                pltpu.async_copy(g_hbm.at[c].at[ebuf.at[par, h * 8 + j, 0]],
                                 rows.at[j], gsem.at[j])
            for j in range(8):
                pltpu.make_async_copy(g_hbm.at[c].at[pl.ds(0, 128)], rows.at[j],
                                      gsem.at[j]).wait()
                pltpu.async_copy(rows.at[j], acc.at[ebuf.at[par, h * 8 + j, 1]],
                                 ssem.at[j], add=True)
        return 0

    lax.fori_loop(0, NSUPER, body, 0)
    for j in range(8):
        _wait_scatter(j)
    plsc.subcore_barrier()

    for k in range(4):
        off = s * OUT_SLICE + k * (OUT_SLICE // 4)
        pltpu.sync_copy(acc.at[pl.ds(off, OUT_SLICE // 4)],
                        out_hbm.at[c, pl.ds(off, OUT_SLICE // 4), :])


def _dot(a, b):
    return jax.lax.dot_general(a, b, (((1,), (0,)), ((), ())),
                               preferred_element_type=jnp.float32)


def _wspec(shape):
    return pl.BlockSpec(shape, lambda i: tuple(0 for _ in shape))


def _pre_body(x_ref, deg_ref, bdPre, prebT, bdFc1, fc1bT, bdFc2, fc2bT,
              bdGcnA, bdGcnB, bdDenseT, densebT, bdF1T, f1bT,
              dinv_ref, based_ref, basef_ref, g2_ref):
    x1 = jax.nn.sigmoid(_dot(x_ref[...], bdPre[...]) + prebT[...])
    ni = jax.nn.relu(_dot(x1, bdFc1[...]) + fc1bT[...])
    h0 = jax.nn.relu(_dot(x1, bdFc2[...]) + fc2bT[...])
    db = deg_ref[...]
    dinv = lax.rsqrt(db[0] + db[1] + 1.0)
    dinv_ref[...] = dinv
    based_ref[...] = _dot(ni, bdDenseT[...]) + densebT[...]
    basef_ref[...] = _dot(ni, bdF1T[...]) + f1bT[...]
    g2_ref[0] = _dot(h0, bdGcnA[...]) * dinv
    g2_ref[1] = _dot(h0, bdGcnB[...]) * dinv


def _layer_body(acc_ref, g_ref, dinv_ref, based_ref, biasA, biasB,
                bdDenseA, bdDenseB, bdGcnA, bdGcnB, gout_ref):
    a = acc_ref[...]
    gb = g_ref[...]
    dinv = dinv_ref[...]
    goA = jax.nn.relu(dinv * (a[0] + gb[0]) + biasA[...])
    goB = jax.nn.relu(dinv * (a[1] + gb[1]) + biasB[...])
    h = jax.nn.relu(_dot(goA, bdDenseA[...]) + _dot(goB, bdDenseB[...])
                    + based_ref[...])
    gout_ref[0] = _dot(h, bdGcnA[...]) * dinv
    gout_ref[1] = _dot(h, bdGcnB[...]) * dinv


def _final_body(acc_ref, g_ref, dinv_ref, based_ref, basef_ref, biasA, biasB,
                bdDenseA, bdDenseB, bdF1, bdF2, f2bt, out_ref):
    a = acc_ref[...]
    gb = g_ref[...]
    dinv = dinv_ref[...]
    goA = jax.nn.relu(dinv * (a[0] + gb[0]) + biasA[...])
    goB = jax.nn.relu(dinv * (a[1] + gb[1]) + biasB[...])
    h = jax.nn.relu(_dot(goA, bdDenseA[...]) + _dot(goB, bdDenseB[...])
                    + based_ref[...])
    h2 = jax.nn.relu(_dot(h, bdF1[...]) + basef_ref[...])
    out_ref[...] = _dot(h2, bdF2[...]) + f2bt[...]


PR = ACCR // 8
PBLK = BLK // 8
PGRID = GRID

_phalf_spec = pl.BlockSpec((NC, PBLK, 128), lambda i: (0, i, 0))
_prow_spec = lambda w: pl.BlockSpec((PBLK, w), lambda i: (i, 0))
_half_spec = pl.BlockSpec((NC, BLK, HALF), lambda i: (0, i, 0))
_row_spec = lambda w: pl.BlockSpec((BLK, w), lambda i: (i, 0))


def _k_pre(x8, deg8, bdPre, prebT, bdFc1, fc1bT, bdFc2, fc2bT, bdGcnA,
           bdGcnB, bdDenseT, densebT, bdF1T, f1bT):
    return pl.pallas_call(
        _pre_body,
        grid=(GRID,),
        in_specs=[
            _prow_spec(48), _phalf_spec,
            _wspec((48, 80)), _wspec((1, 80)),
            _wspec((80, 256)), _wspec((1, 256)),
            _wspec((80, 256)), _wspec((1, 256)),
            _wspec((256, 128)), _wspec((256, 128)),
            _wspec((256, 256)), _wspec((1, 256)),
            _wspec((256, 256)), _wspec((1, 256)),
        ],
        out_specs=[_prow_spec(128), _prow_spec(256), _prow_spec(256),
                   _phalf_spec],
        out_shape=[
            jax.ShapeDtypeStruct((PR, 128), jnp.float32),
            jax.ShapeDtypeStruct((PR, 256), jnp.float32),
            jax.ShapeDtypeStruct((PR, 256), jnp.float32),
            jax.ShapeDtypeStruct((NC, PR, 128), jnp.float32),
        ],
    )(x8, deg8, bdPre, prebT, bdFc1, fc1bT, bdFc2, fc2bT, bdGcnA, bdGcnB,
      bdDenseT, densebT, bdF1T, f1bT)


def _k_layer(acc8, g8, dinv8, based8, biasA, biasB, bdDenseA, bdDenseB,
             bdGcnA, bdGcnB):
    return pl.pallas_call(
        _layer_body,
        grid=(PGRID,),
        in_specs=[
            _phalf_spec, _phalf_spec, _prow_spec(128), _prow_spec(256),
            _wspec((1, 128)), _wspec((1, 128)),
            _wspec((128, 256)), _wspec((128, 256)),
            _wspec((256, 128)), _wspec((256, 128)),
        ],
        out_specs=_phalf_spec,
        out_shape=jax.ShapeDtypeStruct((NC, PR, 128), jnp.float32),
    )(acc8, g8, dinv8, based8, biasA, biasB, bdDenseA, bdDenseB, bdGcnA,
      bdGcnB)


def _k_final(acc8, g8, dinv8, based8, basef8, biasA, biasB, bdDenseA,
             bdDenseB, bdF1, bdF2, f2bt):
    return pl.pallas_call(
        _final_body,
        grid=(PGRID,),
        in_specs=[
            _phalf_spec, _phalf_spec, _prow_spec(128), _prow_spec(256),
            _prow_spec(256),
            _wspec((1, 128)), _wspec((1, 128)),
            _wspec((128, 256)), _wspec((128, 256)),
            _wspec((256, 256)), _wspec((256, 16)), _wspec((1, 16)),
        ],
        out_specs=_prow_spec(16),
        out_shape=jax.ShapeDtypeStruct((PR, 16), jnp.float32),
    )(acc8, g8, dinv8, based8, basef8, biasA, biasB, bdDenseA, bdDenseB,
      bdF1, bdF2, f2bt)


def _block_diag8(W):
    a, b = W.shape
    out = jnp.zeros((8 * a, 8 * b), jnp.float32)
    for j in range(8):
        out = out.at[j * a:(j + 1) * a, j * b:(j + 1) * b].set(W)
    return out


def kernel(x, edge_index, pre_W, pre_b, fc1_W, fc1_b, fc2_W, fc2_b,
           gcn_W, gcn_b, dense_W, dense_b, f1_W, f1_b, f2_W, f2_b):
    src = edge_index[0]
    dst = edge_index[1]
    pad = EPAD - E
    src3 = jnp.concatenate([src, jnp.zeros((pad,), jnp.int32)]).reshape(CROWS, 128)
    dst3 = jnp.concatenate([dst, jnp.full((pad,), DUMMY, jnp.int32)]).reshape(CROWS, 128)
    edges = jnp.stack([src3, dst3], axis=1)

    degout = _sc_degree(dst3)
    deg8 = degout.reshape(NC, PR, 128)

    biasA = jnp.tile(gcn_b[0:HALF], 8).reshape(1, 128)
    biasB = jnp.tile(gcn_b[HALF:D], 8).reshape(1, 128)
    bdDenseA = _block_diag8(dense_W[D:D + HALF])
    bdDenseB = _block_diag8(dense_W[D + HALF:2 * D])
    bdGcnA = _block_diag8(gcn_W[:, 0:HALF])
    bdGcnB = _block_diag8(gcn_W[:, HALF:D])
    bdF1 = _block_diag8(f1_W[D:2 * D])
    bdF2 = _block_diag8(f2_W)
    f2bt = jnp.tile(f2_b, 8).reshape(1, 16)

    bdPre = _block_diag8(pre_W)
    bdFc1 = _block_diag8(fc1_W)
    bdFc2 = _block_diag8(fc2_W)
    bdDenseT = _block_diag8(dense_W[0:D])
    bdF1T = _block_diag8(f1_W[0:D])

    x8 = x.reshape(N // 8, 48)

    dinv8, based8, basef8, g8 = _k_pre(
        x8, deg8, bdPre, jnp.tile(pre_b, 8).reshape(1, -1),
        bdFc1, jnp.tile(fc1_b, 8).reshape(1, -1),
        bdFc2, jnp.tile(fc2_b, 8).reshape(1, -1),
        bdGcnA, bdGcnB, bdDenseT, jnp.tile(dense_b, 8).reshape(1, -1),
        bdF1T, jnp.tile(f1_b, 8).reshape(1, -1))

    for layer in range(6):
        acc2 = _sc_scatter(edges, g8.reshape(NC, ACCR, HALF))
        acc8 = acc2.reshape(NC, PR, 128)
        if layer < 5:
            g8 = _k_layer(acc8, g8, dinv8, based8, biasA, biasB, bdDenseA,
                          bdDenseB, bdGcnA, bdGcnB)
        else:
            out8 = _k_final(acc8, g8, dinv8, based8, basef8, biasA, biasB,
                            bdDenseA, bdDenseB, bdF1, bdF2, f2bt)
            out = out8.reshape(ACCR, 2)[0:N]
    return out
